# Initial kernel scaffold; baseline (speedup 1.0000x reference)
import jax
import jax.numpy as jnp
from jax.experimental import pallas as pl


def kernel(x, W1, W2, current_model_probs):
    raise NotImplementedError("write your pallas kernel here")


# speedup vs baseline: 10.5170x; 10.5170x over previous
import jax
import jax.numpy as jnp
import numpy as np
from jax.experimental import pallas as pl
from jax.experimental.pallas import tpu as pltpu

_E_MARGIN = float(np.log(1000) / 2 - 1)
_D_MARGIN = 0.05
_CORESET = 64
_N = 128


def _eata_tc_kernel(x_ref, w1_ref, w2_ref, cmp_ref,
                    out_ref, loss_ref, up_ref, feats_ref):
    feats = jnp.dot(x_ref[...], w1_ref[...], preferred_element_type=jnp.float32)
    feats_ref[...] = feats
    outs = jnp.dot(feats, w2_ref[...], preferred_element_type=jnp.float32)
    out_ref[...] = outs

    m = jnp.max(outs, axis=1, keepdims=True)
    e = jnp.exp(outs - m)
    s = jnp.sum(e, axis=1, keepdims=True)
    probs = e / s
    logp = (outs - m) - jnp.log(s)
    ent = -jnp.sum(probs * logp, axis=1, keepdims=True)

    cmp = cmp_ref[...]
    cos_num = jnp.sum(probs * cmp, axis=1, keepdims=True)
    pn = jnp.sqrt(jnp.sum(probs * probs, axis=1, keepdims=True))
    cn = jnp.sqrt(jnp.sum(cmp * cmp))
    cos = cos_num / (pn * cn + 1e-8)

    m2 = (ent < _E_MARGIN) & (jnp.abs(cos) < _D_MARGIN)
    m2f = m2.astype(jnp.float32)
    n_sel = jnp.sum(m2.astype(jnp.int32))
    k = jnp.minimum(_CORESET, n_sel)
    kf = jnp.maximum(k, 1).astype(jnp.float32)
    n_valid = jnp.maximum(n_sel, 1).astype(jnp.float32)
    mu = jnp.sum(feats * m2f, axis=0, keepdims=True) / n_valid

    def body(i, state):
        sel_mask, mu_t = state
        t = i + 1
        tf = t.astype(jnp.float32)
        v = tf * mu - (tf - 1.0) * mu_t
        scores = jnp.sum(feats * v, axis=1, keepdims=True)
        avail = m2 & (sel_mask == 0.0)
        scores = jnp.where(avail, scores, -jnp.inf)
        ti = jnp.argmax(scores[:, 0]).astype(jnp.int32)
        active = t <= k
        onehot = (jax.lax.broadcasted_iota(jnp.int32, (_N, 1), 0) == ti
                  ).astype(jnp.float32)
        sel_mask = jnp.where(active, sel_mask + onehot, sel_mask)
        x_t = feats_ref[pl.ds(ti, 1), :]
        mu_t = jnp.where(active, mu_t + (x_t - mu_t) / tf, mu_t)
        return sel_mask, mu_t

    sel0 = jnp.zeros((_N, 1), jnp.float32)
    mu_t0 = jnp.zeros_like(mu)
    sel_mask, _ = jax.lax.fori_loop(0, _CORESET, body, (sel0, mu_t0))

    mean_probs = jnp.sum(probs * sel_mask, axis=0, keepdims=True) / kf
    updated = jnp.where(k > 0, 0.9 * cmp + 0.1 * mean_probs, cmp)
    up_ref[...] = updated
    coeff = jnp.exp(_E_MARGIN - ent)
    loss = jnp.sum(ent * coeff * sel_mask) / kf
    loss_ref[...] = jnp.where(k > 0, loss, 0.0)[None, None]


def kernel(x, W1, W2, current_model_probs):
    n, c = x.shape[0], W2.shape[1]
    outs, loss, updated = pl.pallas_call(
        _eata_tc_kernel,
        out_shape=[
            jax.ShapeDtypeStruct((n, c), jnp.float32),
            jax.ShapeDtypeStruct((1, 1), jnp.float32),
            jax.ShapeDtypeStruct((1, c), jnp.float32),
        ],
        scratch_shapes=[pltpu.VMEM((n, W1.shape[1]), jnp.float32)],
    )(x, W1, W2, current_model_probs.reshape(1, c))
    return outs, loss.reshape(()), updated.reshape(c)


# speedup vs baseline: 14.5856x; 1.3869x over previous
import jax
import jax.numpy as jnp
import numpy as np
from jax.experimental import pallas as pl
from jax.experimental.pallas import tpu as pltpu

_E_MARGIN = float(np.log(1000) / 2 - 1)
_D_MARGIN = 0.05
_CORESET = 64
_N = 128


def _eata_tc_kernel(x_ref, w1_ref, w2_ref, cmp_ref,
                    out_ref, loss_ref, up_ref, gram_ref):
    feats = jnp.dot(x_ref[...], w1_ref[...], preferred_element_type=jnp.float32)
    outs = jnp.dot(feats, w2_ref[...], preferred_element_type=jnp.float32)
    out_ref[...] = outs

    m = jnp.max(outs, axis=1, keepdims=True)
    e = jnp.exp(outs - m)
    s = jnp.sum(e, axis=1, keepdims=True)
    probs = e / s
    logp = (outs - m) - jnp.log(s)
    ent = -jnp.sum(probs * logp, axis=1, keepdims=True)

    cmp = cmp_ref[...]
    cos_num = jnp.sum(probs * cmp, axis=1, keepdims=True)
    pn = jnp.sqrt(jnp.sum(probs * probs, axis=1, keepdims=True))
    cn = jnp.sqrt(jnp.sum(cmp * cmp))
    cos = cos_num / (pn * cn + 1e-8)

    m2 = (ent < _E_MARGIN) & (jnp.abs(cos) < _D_MARGIN)
    m2f = m2.astype(jnp.float32)
    n_sel = jnp.sum(m2.astype(jnp.int32))
    k = jnp.minimum(_CORESET, n_sel)
    kf = jnp.maximum(k, 1).astype(jnp.float32)
    n_valid = jnp.maximum(n_sel, 1).astype(jnp.float32)

    gram = jax.lax.dot_general(feats, feats, (((1,), (1,)), ((), ())),
                               preferred_element_type=jnp.float32)
    gram_ref[...] = gram
    m2row = jnp.transpose(m2f)
    g0 = jax.lax.dot_general(m2row, gram, (((1,), (0,)), ((), ())),
                             preferred_element_type=jnp.float32) / n_valid

    lane = jax.lax.broadcasted_iota(jnp.int32, (1, _N), 1)

    def body(i, state):
        sel, h = state
        t = i + 1
        tf = t.astype(jnp.float32)
        scores = tf * g0 - (tf - 1.0) * h
        avail = (m2row > 0.0) & (sel == 0.0)
        scores = jnp.where(avail, scores, -jnp.inf)
        ti = jnp.argmax(scores[0, :]).astype(jnp.int32)
        active = t <= k
        onehot = (lane == ti).astype(jnp.float32)
        sel = jnp.where(active, sel + onehot, sel)
        grow = gram_ref[pl.ds(ti, 1), :]
        h = jnp.where(active, h + (grow - h) / tf, h)
        return sel, h

    sel0 = jnp.zeros((1, _N), jnp.float32)
    sel, _ = jax.lax.fori_loop(0, _CORESET, body, (sel0, sel0))

    mean_probs = jax.lax.dot_general(sel, probs, (((1,), (0,)), ((), ())),
                                     preferred_element_type=jnp.float32) / kf
    updated = jnp.where(k > 0, 0.9 * cmp + 0.1 * mean_probs, cmp)
    up_ref[...] = updated
    entcoeff = ent * jnp.exp(_E_MARGIN - ent)
    loss = jax.lax.dot_general(sel, entcoeff, (((1,), (0,)), ((), ())),
                               preferred_element_type=jnp.float32) / kf
    loss_ref[...] = jnp.where(k > 0, loss, 0.0)


def kernel(x, W1, W2, current_model_probs):
    n, c = x.shape[0], W2.shape[1]
    outs, loss, updated = pl.pallas_call(
        _eata_tc_kernel,
        out_shape=[
            jax.ShapeDtypeStruct((n, c), jnp.float32),
            jax.ShapeDtypeStruct((1, 1), jnp.float32),
            jax.ShapeDtypeStruct((1, c), jnp.float32),
        ],
        scratch_shapes=[pltpu.VMEM((n, n), jnp.float32)],
    )(x, W1, W2, current_model_probs.reshape(1, c))
    return outs, loss.reshape(()), updated.reshape(c)
